# Initial kernel scaffold; baseline (speedup 1.0000x reference)
import jax
import jax.numpy as jnp
from jax.experimental import pallas as pl


def kernel(x, idx, val):
    raise NotImplementedError("write your pallas kernel here")


# speedup vs baseline: 3.1772x; 3.1772x over previous
import functools

import jax
import jax.numpy as jnp
from jax import lax
from jax.experimental import pallas as pl
from jax.experimental.pallas import tpu as pltpu
from jax.experimental.pallas import tpu_sc as plsc

N_ROWS = 16384
N_COLS = 1024
N_SITES = 4096

_BR = 512
_NBLK = N_ROWS // _BR

_NC, _NS, _L = 2, 16, 16
_NW = _NC * _NS
_K = N_SITES // _NW


def _copy_minmax_body(x_ref, out_ref, mnmx_ref):
    i = pl.program_id(0)
    blk = x_ref[...]
    out_ref[...] = blk
    bmin = jnp.min(blk)
    bmax = jnp.max(blk)

    @pl.when(i == 0)
    def _init():
        mnmx_ref[0:1, :] = jnp.full((1, 128), bmin, jnp.float32)
        mnmx_ref[1:2, :] = jnp.full((1, 128), bmax, jnp.float32)

    @pl.when(i > 0)
    def _acc():
        mnmx_ref[0:1, :] = jnp.minimum(mnmx_ref[0:1, :], bmin)
        mnmx_ref[1:2, :] = jnp.maximum(mnmx_ref[1:2, :], bmax)


_copy_minmax = pl.pallas_call(
    _copy_minmax_body,
    grid=(_NBLK,),
    in_specs=[pl.BlockSpec((_BR, N_COLS), lambda i: (i, 0))],
    out_specs=[
        pl.BlockSpec((_BR, N_COLS), lambda i: (i, 0)),
        pl.BlockSpec((2, 128), lambda i: (0, 0)),
    ],
    out_shape=[
        jax.ShapeDtypeStruct((N_ROWS, N_COLS), jnp.float32),
        jax.ShapeDtypeStruct((2, 128), jnp.float32),
    ],
)


def _sc_scatter_body(out_ref, idx_hbm, val_hbm, mnmx_hbm,
                     idx_v, val_v, inj_v, mn_row, mx_row, sem):
    wid = lax.axis_index("s") * _NC + lax.axis_index("c")
    base = wid * _K
    pltpu.sync_copy(idx_hbm.at[pl.ds(base, _K)], idx_v)
    pltpu.sync_copy(val_hbm.at[pl.ds(base, _K)], val_v)
    pltpu.sync_copy(mnmx_hbm.at[0], mn_row)
    pltpu.sync_copy(mnmx_hbm.at[1], mx_row)
    mn = mn_row[pl.ds(0, _L)]
    scale = mx_row[pl.ds(0, _L)] - mn
    for j in range(_K // _L):
        s = pl.ds(j * _L, _L)
        inj_v[s] = mn + val_v[s] * scale
    pltpu.async_copy(inj_v, out_ref.at[idx_v], sem).wait()


@functools.cache
def _get_sc_scatter():
    return pl.kernel(
        _sc_scatter_body,
        out_type=(),
        mesh=plsc.VectorSubcoreMesh(core_axis_name="c", subcore_axis_name="s"),
        scratch_types=[
            pltpu.VMEM((_K,), jnp.int32),
            pltpu.VMEM((_K,), jnp.float32),
            pltpu.VMEM((_K,), jnp.float32),
            pltpu.VMEM((128,), jnp.float32),
            pltpu.VMEM((128,), jnp.float32),
            pltpu.SemaphoreType.DMA,
        ],
    )


def kernel(x, idx, val):
    idx32 = idx.astype(jnp.int32)
    out, mnmx = _copy_minmax(x)
    ref = jax.new_ref(out.reshape(-1))
    _get_sc_scatter()(ref, idx32, val, mnmx)
    return jax.freeze(ref).reshape(N_ROWS, N_COLS)
